# baseline (speedup 1.0000x reference)
import functools

import jax
import jax.numpy as jnp
from jax import lax
from jax.experimental import pallas as pl
from jax.experimental.pallas import tpu as pltpu
from jax.experimental.pallas import tpu_sc as plsc

B, C, H, W = 4, 128, 256, 512
P = 49152
BC = B * C
NC, NS, L = 2, 16, 16
NW = NC * NS
PPW = P // NW
CHUNK = 16
NCHUNK = PPW // CHUNK
NSLICE = BC // L


UNROLL = 8


def _sc_body(xt_hbm, index_hbm, out_hbm, ibuf, gbuf, obuf, gsems, osems):
    wid = lax.axis_index("s") * NC + lax.axis_index("c")
    base = wid * PPW

    pltpu.sync_copy(index_hbm.at[:, pl.ds(base, PPW)], ibuf)

    def fire_gather(g, slot):
        o = g * CHUNK
        x0 = ibuf[0, pl.ds(o, L)].astype(jnp.int32)
        y0 = ibuf[1, pl.ds(o, L)].astype(jnp.int32)
        x1 = ibuf[2, pl.ds(o, L)].astype(jnp.int32)
        y1 = ibuf[3, pl.ds(o, L)].astype(jnp.int32)
        r0 = y0 * W
        r1 = y1 * W
        sem = gsems[slot]
        pltpu.async_copy(xt_hbm.at[r0 + x0], gbuf.at[slot, pl.ds(0, L)], sem)
        pltpu.async_copy(xt_hbm.at[r1 + x0], gbuf.at[slot, pl.ds(L, L)], sem)
        pltpu.async_copy(xt_hbm.at[r0 + x1], gbuf.at[slot, pl.ds(2 * L, L)], sem)
        pltpu.async_copy(xt_hbm.at[r1 + x1], gbuf.at[slot, pl.ds(3 * L, L)], sem)

    def wait_gather(slot):
        pltpu.make_async_copy(
            xt_hbm.at[pl.ds(0, 4 * L)], gbuf.at[slot], gsems[slot]
        ).wait()

    def drain_out(slot):
        pltpu.make_async_copy(
            obuf.at[slot], out_hbm.at[pl.ds(0, CHUNK)], osems[slot]
        ).wait()

    def do_chunk(g, k, slot):
        wait_gather(slot)
        @pl.when(k > 0)
        def _():
            drain_out(slot)

        o = g * CHUNK
        wav = ibuf[4, pl.ds(o, L)]
        wbv = ibuf[5, pl.ds(o, L)]
        wcv = ibuf[6, pl.ds(o, L)]
        wdv = ibuf[7, pl.ds(o, L)]

        for i in range(CHUNK):
            wa = wav[i]
            wb = wbv[i]
            wc = wcv[i]
            wd = wdv[i]

            def blend(s, _):
                for u in range(UNROLL):
                    col = (s * UNROLL + u) * L
                    va = gbuf[slot, i, pl.ds(col, L)]
                    vb = gbuf[slot, L + i, pl.ds(col, L)]
                    vc = gbuf[slot, 2 * L + i, pl.ds(col, L)]
                    vd = gbuf[slot, 3 * L + i, pl.ds(col, L)]
                    obuf[slot, i, pl.ds(col, L)] = (
                        wa * va + wb * vb + wc * vc + wd * vd
                    )
                return 0

            lax.fori_loop(0, NSLICE // UNROLL, blend, 0)

        pltpu.async_copy(
            obuf.at[slot], out_hbm.at[pl.ds(base + o, CHUNK)], osems[slot]
        )
        fire_gather(jnp.minimum(g + 2, NCHUNK - 1), slot)

    fire_gather(0, 0)
    fire_gather(1, 1)

    def body(k, _):
        g = 2 * k
        do_chunk(g, k, 0)
        do_chunk(g + 1, k, 1)
        return 0

    lax.fori_loop(0, NCHUNK // 2, body, 0)
    wait_gather(0)
    wait_gather(1)
    drain_out(0)
    drain_out(1)


def _tc_transpose_body(inb, outb):
    outb[...] = inb[...].T


def _tc_transpose(out_t):
    TP = 512
    return pl.pallas_call(
        _tc_transpose_body,
        grid=(P // TP, BC // BC),
        in_specs=[pl.BlockSpec((TP, BC), lambda i, j: (i, j))],
        out_specs=pl.BlockSpec((BC, TP), lambda i, j: (j, i)),
        out_shape=jax.ShapeDtypeStruct((BC, P), jnp.float32),
    )(out_t)


@jax.jit
def kernel(x, index):
    xt = x.transpose(2, 3, 0, 1).reshape(H * W, BC)
    mesh = plsc.VectorSubcoreMesh(
        core_axis_name="c", subcore_axis_name="s", num_cores=NC, num_subcores=NS
    )
    out_t = pl.kernel(
        _sc_body,
        out_type=jax.ShapeDtypeStruct((P, BC), jnp.float32),
        mesh=mesh,
        scratch_types=[
            pltpu.VMEM((8, PPW), jnp.float32),
            pltpu.VMEM((2, 4 * L, BC), jnp.float32),
            pltpu.VMEM((2, CHUNK, BC), jnp.float32),
            [pltpu.SemaphoreType.DMA, pltpu.SemaphoreType.DMA],
            [pltpu.SemaphoreType.DMA, pltpu.SemaphoreType.DMA],
        ],
    )(xt, index)
    return _tc_transpose(out_t).reshape(B, C, 1, P)


# speedup vs baseline: 1.0296x; 1.0296x over previous
import functools

import jax
import jax.numpy as jnp
from jax import lax
from jax.experimental import pallas as pl
from jax.experimental.pallas import tpu as pltpu
from jax.experimental.pallas import tpu_sc as plsc

B, C, H, W = 4, 128, 256, 512
P = 49152
BC = B * C
NC, NS, L = 2, 16, 16
NW = NC * NS
PPW = P // NW
CHUNK = 16
NCHUNK = PPW // CHUNK
NSLICE = BC // L


UNROLL = 8


def _sc_body(xt_hbm, index_hbm, out_hbm, ibuf, gbuf, obuf, gsems, osems):
    wid = lax.axis_index("s") * NC + lax.axis_index("c")
    base = wid * PPW

    pltpu.sync_copy(index_hbm.at[:, pl.ds(base, PPW)], ibuf)

    def fire_gather(g, slot):
        o = g * CHUNK
        x0 = ibuf[0, pl.ds(o, L)].astype(jnp.int32)
        y0 = ibuf[1, pl.ds(o, L)].astype(jnp.int32)
        x1 = ibuf[2, pl.ds(o, L)].astype(jnp.int32)
        y1 = ibuf[3, pl.ds(o, L)].astype(jnp.int32)
        r0 = y0 * W
        r1 = y1 * W
        sem = gsems[slot]
        pltpu.async_copy(xt_hbm.at[r0 + x0], gbuf.at[slot, pl.ds(0, L)], sem)
        pltpu.async_copy(xt_hbm.at[r1 + x0], gbuf.at[slot, pl.ds(L, L)], sem)
        pltpu.async_copy(xt_hbm.at[r0 + x1], gbuf.at[slot, pl.ds(2 * L, L)], sem)
        pltpu.async_copy(xt_hbm.at[r1 + x1], gbuf.at[slot, pl.ds(3 * L, L)], sem)

    def wait_gather(slot):
        pltpu.make_async_copy(
            xt_hbm.at[pl.ds(0, 4 * L)], gbuf.at[slot], gsems[slot]
        ).wait()

    def drain_out(slot):
        pltpu.make_async_copy(
            obuf.at[slot], out_hbm.at[pl.ds(0, CHUNK)], osems[slot]
        ).wait()

    def do_chunk(g, k, slot):
        wait_gather(slot)
        @pl.when(k > 0)
        def _():
            drain_out(slot)

        o = g * CHUNK
        wav = ibuf[4, pl.ds(o, L)]
        wbv = ibuf[5, pl.ds(o, L)]
        wcv = ibuf[6, pl.ds(o, L)]
        wdv = ibuf[7, pl.ds(o, L)]

        for i in range(CHUNK):
            wa = wav[i]
            wb = wbv[i]
            wc = wcv[i]
            wd = wdv[i]

            def blend(s, _):
                for u in range(UNROLL):
                    col = (s * UNROLL + u) * L
                    va = gbuf[slot, i, pl.ds(col, L)]
                    vb = gbuf[slot, L + i, pl.ds(col, L)]
                    vc = gbuf[slot, 2 * L + i, pl.ds(col, L)]
                    vd = gbuf[slot, 3 * L + i, pl.ds(col, L)]
                    obuf[slot, i, pl.ds(col, L)] = (
                        wa * va + wb * vb + wc * vc + wd * vd
                    )
                return 0

            lax.fori_loop(0, NSLICE // UNROLL, blend, 0)

        pltpu.async_copy(
            obuf.at[slot], out_hbm.at[pl.ds(base + o, CHUNK)], osems[slot]
        )
        fire_gather(jnp.minimum(g + 2, NCHUNK - 1), slot)

    fire_gather(0, 0)
    fire_gather(1, 1)

    def body(k, _):
        g = 2 * k
        do_chunk(g, k, 0)
        do_chunk(g + 1, k, 1)
        return 0

    lax.fori_loop(0, NCHUNK // 2, body, 0)
    wait_gather(0)
    wait_gather(1)
    drain_out(0)
    drain_out(1)


def _tc_transpose_body(inb, outb):
    outb[...] = inb[...].T


def _tc_transpose(out_t):
    TP = 512
    return pl.pallas_call(
        _tc_transpose_body,
        grid=(P // TP, BC // BC),
        in_specs=[pl.BlockSpec((TP, BC), lambda i, j: (i, j))],
        out_specs=pl.BlockSpec((BC, TP), lambda i, j: (j, i)),
        out_shape=jax.ShapeDtypeStruct((BC, P), jnp.float32),
    )(out_t)


@jax.jit
def kernel(x, index):
    xt = x.reshape(BC, H * W).T
    mesh = plsc.VectorSubcoreMesh(
        core_axis_name="c", subcore_axis_name="s", num_cores=NC, num_subcores=NS
    )
    out_t = pl.kernel(
        _sc_body,
        out_type=jax.ShapeDtypeStruct((P, BC), jnp.float32),
        mesh=mesh,
        scratch_types=[
            pltpu.VMEM((8, PPW), jnp.float32),
            pltpu.VMEM((2, 4 * L, BC), jnp.float32),
            pltpu.VMEM((2, CHUNK, BC), jnp.float32),
            [pltpu.SemaphoreType.DMA, pltpu.SemaphoreType.DMA],
            [pltpu.SemaphoreType.DMA, pltpu.SemaphoreType.DMA],
        ],
    )(xt, index)
    return out_t.T.reshape(B, C, 1, P)


# speedup vs baseline: 1.0427x; 1.0127x over previous
import functools

import jax
import jax.numpy as jnp
from jax import lax
from jax.experimental import pallas as pl
from jax.experimental.pallas import tpu as pltpu
from jax.experimental.pallas import tpu_sc as plsc

B, C, H, W = 4, 128, 256, 512
P = 49152
BC = B * C
NC, NS, L = 2, 16, 16
NW = NC * NS
PPW = P // NW
CHUNK = 16
NCHUNK = PPW // CHUNK
NSLICE = BC // L


UNROLL = 8


def _sc_body(xt_hbm, index_hbm, out_hbm, ibuf, gbuf, obuf, gsems, osems):
    wid = lax.axis_index("s") * NC + lax.axis_index("c")
    base = wid * PPW

    pltpu.sync_copy(index_hbm.at[:, pl.ds(base, PPW)], ibuf)

    def fire_gather(g, slot):
        o = g * CHUNK
        x0 = ibuf[0, pl.ds(o, L)].astype(jnp.int32)
        y0 = ibuf[1, pl.ds(o, L)].astype(jnp.int32)
        x1 = ibuf[2, pl.ds(o, L)].astype(jnp.int32)
        y1 = ibuf[3, pl.ds(o, L)].astype(jnp.int32)
        r0 = y0 * W
        r1 = y1 * W
        sem = gsems[slot]
        pltpu.async_copy(xt_hbm.at[r0 + x0], gbuf.at[slot, pl.ds(0, L)], sem)
        pltpu.async_copy(xt_hbm.at[r1 + x0], gbuf.at[slot, pl.ds(L, L)], sem)
        pltpu.async_copy(xt_hbm.at[r0 + x1], gbuf.at[slot, pl.ds(2 * L, L)], sem)
        pltpu.async_copy(xt_hbm.at[r1 + x1], gbuf.at[slot, pl.ds(3 * L, L)], sem)

    def wait_gather(slot):
        pltpu.make_async_copy(
            xt_hbm.at[pl.ds(0, 4 * L)], gbuf.at[slot], gsems[slot]
        ).wait()

    def drain_out(slot):
        pltpu.make_async_copy(
            obuf.at[slot], out_hbm.at[pl.ds(0, CHUNK)], osems[slot]
        ).wait()

    def do_chunk(g, k, slot):
        wait_gather(slot)
        @pl.when(k > 0)
        def _():
            drain_out(slot)

        o = g * CHUNK
        wav = ibuf[4, pl.ds(o, L)]
        wbv = ibuf[5, pl.ds(o, L)]
        wcv = ibuf[6, pl.ds(o, L)]
        wdv = ibuf[7, pl.ds(o, L)]

        for i in range(CHUNK):
            wa = wav[i]
            wb = wbv[i]
            wc = wcv[i]
            wd = wdv[i]

            def blend(s, _):
                for u in range(UNROLL):
                    col = (s * UNROLL + u) * L
                    va = gbuf[slot, i, pl.ds(col, L)]
                    vb = gbuf[slot, L + i, pl.ds(col, L)]
                    vc = gbuf[slot, 2 * L + i, pl.ds(col, L)]
                    vd = gbuf[slot, 3 * L + i, pl.ds(col, L)]
                    obuf[slot, i, pl.ds(col, L)] = (
                        wa * va + wb * vb + wc * vc + wd * vd
                    )
                return 0

            lax.fori_loop(0, NSLICE // UNROLL, blend, 0)

        pltpu.async_copy(
            obuf.at[slot], out_hbm.at[pl.ds(base + o, CHUNK)], osems[slot]
        )
        fire_gather(jnp.minimum(g + 2, NCHUNK - 1), slot)

    fire_gather(0, 0)
    fire_gather(1, 1)

    def body(k, _):
        g = 2 * k
        do_chunk(g, k, 0)
        do_chunk(g + 1, k, 1)
        return 0

    lax.fori_loop(0, NCHUNK // 2, body, 0)
    wait_gather(0)
    wait_gather(1)
    drain_out(0)
    drain_out(1)


def _tc_transpose_body(inb, outb):
    outb[...] = inb[...].T


def _tc_transpose(out_t):
    TP = 512
    return pl.pallas_call(
        _tc_transpose_body,
        grid=(P // TP, BC // BC),
        in_specs=[pl.BlockSpec((TP, BC), lambda i, j: (i, j))],
        out_specs=pl.BlockSpec((BC, TP), lambda i, j: (j, i)),
        out_shape=jax.ShapeDtypeStruct((BC, P), jnp.float32),
    )(out_t)


@jax.jit
def kernel(x, index):
    xt = x.reshape(BC, H * W, 1).transpose(1, 2, 0).reshape(H * W, BC)
    mesh = plsc.VectorSubcoreMesh(
        core_axis_name="c", subcore_axis_name="s", num_cores=NC, num_subcores=NS
    )
    out_t = pl.kernel(
        _sc_body,
        out_type=jax.ShapeDtypeStruct((P, BC), jnp.float32),
        mesh=mesh,
        scratch_types=[
            pltpu.VMEM((8, PPW), jnp.float32),
            pltpu.VMEM((2, 4 * L, BC), jnp.float32),
            pltpu.VMEM((2, CHUNK, BC), jnp.float32),
            [pltpu.SemaphoreType.DMA, pltpu.SemaphoreType.DMA],
            [pltpu.SemaphoreType.DMA, pltpu.SemaphoreType.DMA],
        ],
    )(xt, index)
    return out_t.reshape(P, B, C).transpose(1, 2, 0).reshape(B, C, 1, P)


# speedup vs baseline: 1.3573x; 1.3017x over previous
import functools

import jax
import jax.numpy as jnp
from jax import lax
from jax.experimental import pallas as pl
from jax.experimental.pallas import tpu as pltpu
from jax.experimental.pallas import tpu_sc as plsc

B, C, H, W = 4, 128, 256, 512
P = 49152
BC = B * C
NC, NS, L = 2, 16, 16
NW = NC * NS
PPW = P // NW
CHUNK = 16
NCHUNK = PPW // CHUNK
NSLICE = BC // L


UNROLL = 8


def _sc_body(xt_hbm, index_hbm, out_hbm, ibuf, gbuf, obuf, gsems, osems):
    wid = lax.axis_index("s") * NC + lax.axis_index("c")
    base = wid * PPW

    pltpu.sync_copy(index_hbm.at[:, pl.ds(base, PPW)], ibuf)

    def fire_gather(g, slot):
        o = g * CHUNK
        x0 = ibuf[0, pl.ds(o, L)].astype(jnp.int32)
        y0 = ibuf[1, pl.ds(o, L)].astype(jnp.int32)
        x1 = ibuf[2, pl.ds(o, L)].astype(jnp.int32)
        y1 = ibuf[3, pl.ds(o, L)].astype(jnp.int32)
        r0 = y0 * W
        r1 = y1 * W
        sem = gsems[slot]
        pltpu.async_copy(xt_hbm.at[r0 + x0], gbuf.at[slot, pl.ds(0, L)], sem)
        pltpu.async_copy(xt_hbm.at[r1 + x0], gbuf.at[slot, pl.ds(L, L)], sem)
        pltpu.async_copy(xt_hbm.at[r0 + x1], gbuf.at[slot, pl.ds(2 * L, L)], sem)
        pltpu.async_copy(xt_hbm.at[r1 + x1], gbuf.at[slot, pl.ds(3 * L, L)], sem)

    def wait_gather(slot):
        pltpu.make_async_copy(
            xt_hbm.at[pl.ds(0, 4 * L)], gbuf.at[slot], gsems[slot]
        ).wait()

    def drain_out(slot):
        pltpu.make_async_copy(
            obuf.at[slot], out_hbm.at[pl.ds(0, CHUNK)], osems[slot]
        ).wait()

    def do_chunk(g, k, slot):
        wait_gather(slot)
        @pl.when(k > 0)
        def _():
            drain_out(slot)

        o = g * CHUNK
        wav = ibuf[4, pl.ds(o, L)]
        wbv = ibuf[5, pl.ds(o, L)]
        wcv = ibuf[6, pl.ds(o, L)]
        wdv = ibuf[7, pl.ds(o, L)]

        for i in range(CHUNK):
            wa = wav[i]
            wb = wbv[i]
            wc = wcv[i]
            wd = wdv[i]

            def blend(s, _):
                for u in range(UNROLL):
                    col = (s * UNROLL + u) * L
                    va = gbuf[slot, i, pl.ds(col, L)]
                    vb = gbuf[slot, L + i, pl.ds(col, L)]
                    vc = gbuf[slot, 2 * L + i, pl.ds(col, L)]
                    vd = gbuf[slot, 3 * L + i, pl.ds(col, L)]
                    obuf[slot, i, pl.ds(col, L)] = (
                        wa * va + wb * vb + wc * vc + wd * vd
                    )
                return 0

            lax.fori_loop(0, NSLICE // UNROLL, blend, 0)

        pltpu.async_copy(
            obuf.at[slot], out_hbm.at[pl.ds(base + o, CHUNK)], osems[slot]
        )
        fire_gather(jnp.minimum(g + 2, NCHUNK - 1), slot)

    fire_gather(0, 0)
    fire_gather(1, 1)

    def body(k, _):
        g = 2 * k
        do_chunk(g, k, 0)
        do_chunk(g + 1, k, 1)
        return 0

    lax.fori_loop(0, NCHUNK // 2, body, 0)
    wait_gather(0)
    wait_gather(1)
    drain_out(0)
    drain_out(1)


def _tc_transpose_body(inb, outb):
    outb[...] = inb[...].T


def _tc_transpose(out_t):
    TP = 512
    return pl.pallas_call(
        _tc_transpose_body,
        grid=(P // TP, BC // BC),
        in_specs=[pl.BlockSpec((TP, BC), lambda i, j: (i, j))],
        out_specs=pl.BlockSpec((BC, TP), lambda i, j: (j, i)),
        out_shape=jax.ShapeDtypeStruct((BC, P), jnp.float32),
    )(out_t)


@jax.jit
def kernel(x, index):
    xt = x.reshape(BC, H, W).transpose(1, 2, 0).reshape(H * W, BC)
    mesh = plsc.VectorSubcoreMesh(
        core_axis_name="c", subcore_axis_name="s", num_cores=NC, num_subcores=NS
    )
    out_t = pl.kernel(
        _sc_body,
        out_type=jax.ShapeDtypeStruct((P, BC), jnp.float32),
        mesh=mesh,
        scratch_types=[
            pltpu.VMEM((8, PPW), jnp.float32),
            pltpu.VMEM((2, 4 * L, BC), jnp.float32),
            pltpu.VMEM((2, CHUNK, BC), jnp.float32),
            [pltpu.SemaphoreType.DMA, pltpu.SemaphoreType.DMA],
            [pltpu.SemaphoreType.DMA, pltpu.SemaphoreType.DMA],
        ],
    )(xt, index)
    return out_t.reshape(P, B, C).transpose(1, 2, 0).reshape(B, C, 1, P)


# speedup vs baseline: 1.5490x; 1.1412x over previous
import functools

import jax
import jax.numpy as jnp
from jax import lax
from jax.experimental import pallas as pl
from jax.experimental.pallas import tpu as pltpu
from jax.experimental.pallas import tpu_sc as plsc

B, C, H, W = 4, 128, 256, 512
P = 49152
BC = B * C
NC, NS, L = 2, 16, 16
NW = NC * NS
PPW = P // NW
CHUNK = 16
NCHUNK = PPW // CHUNK
NSLICE = BC // L


UNROLL = 8


def _sc_body(xt_hbm, index_hbm, out_hbm, ibuf, gbuf, obuf, gsems, osems):
    wid = lax.axis_index("s") * NC + lax.axis_index("c")
    base = wid * PPW

    pltpu.sync_copy(index_hbm.at[:, pl.ds(base, PPW)], ibuf)

    def fire_gather(g, slot):
        o = g * CHUNK
        x0 = ibuf[0, pl.ds(o, L)].astype(jnp.int32)
        y0 = ibuf[1, pl.ds(o, L)].astype(jnp.int32)
        x1 = ibuf[2, pl.ds(o, L)].astype(jnp.int32)
        y1 = ibuf[3, pl.ds(o, L)].astype(jnp.int32)
        r0 = y0 * W
        r1 = y1 * W
        sem = gsems[slot]
        pltpu.async_copy(xt_hbm.at[r0 + x0], gbuf.at[slot, pl.ds(0, L)], sem)
        pltpu.async_copy(xt_hbm.at[r1 + x0], gbuf.at[slot, pl.ds(L, L)], sem)
        pltpu.async_copy(xt_hbm.at[r0 + x1], gbuf.at[slot, pl.ds(2 * L, L)], sem)
        pltpu.async_copy(xt_hbm.at[r1 + x1], gbuf.at[slot, pl.ds(3 * L, L)], sem)

    def wait_gather(slot):
        pltpu.make_async_copy(
            xt_hbm.at[pl.ds(0, 4 * L)], gbuf.at[slot], gsems[slot]
        ).wait()

    def drain_out(slot):
        for b in range(B):
            pltpu.make_async_copy(
                obuf.at[slot, :, pl.ds(b * C, C)],
                out_hbm.at[b, pl.ds(0, CHUNK)],
                osems[slot],
            ).wait()

    def do_chunk(g, k, slot):
        wait_gather(slot)
        @pl.when(k > 0)
        def _():
            drain_out(slot)

        o = g * CHUNK
        wav = ibuf[4, pl.ds(o, L)]
        wbv = ibuf[5, pl.ds(o, L)]
        wcv = ibuf[6, pl.ds(o, L)]
        wdv = ibuf[7, pl.ds(o, L)]

        for i in range(CHUNK):
            wa = wav[i]
            wb = wbv[i]
            wc = wcv[i]
            wd = wdv[i]

            def blend(s, _):
                for u in range(UNROLL):
                    col = (s * UNROLL + u) * L
                    va = gbuf[slot, i, pl.ds(col, L)]
                    vb = gbuf[slot, L + i, pl.ds(col, L)]
                    vc = gbuf[slot, 2 * L + i, pl.ds(col, L)]
                    vd = gbuf[slot, 3 * L + i, pl.ds(col, L)]
                    obuf[slot, i, pl.ds(col, L)] = (
                        wa * va + wb * vb + wc * vc + wd * vd
                    )
                return 0

            lax.fori_loop(0, NSLICE // UNROLL, blend, 0)

        for b in range(B):
            pltpu.async_copy(
                obuf.at[slot, :, pl.ds(b * C, C)],
                out_hbm.at[b, pl.ds(base + o, CHUNK)],
                osems[slot],
            )
        fire_gather(jnp.minimum(g + 2, NCHUNK - 1), slot)

    fire_gather(0, 0)
    fire_gather(1, 1)

    def body(k, _):
        g = 2 * k
        do_chunk(g, k, 0)
        do_chunk(g + 1, k, 1)
        return 0

    lax.fori_loop(0, NCHUNK // 2, body, 0)
    wait_gather(0)
    wait_gather(1)
    drain_out(0)
    drain_out(1)


def _tc_transpose_body(inb, outb):
    outb[...] = inb[...].T


def _tc_transpose(out_t):
    TP = 512
    return pl.pallas_call(
        _tc_transpose_body,
        grid=(P // TP, BC // BC),
        in_specs=[pl.BlockSpec((TP, BC), lambda i, j: (i, j))],
        out_specs=pl.BlockSpec((BC, TP), lambda i, j: (j, i)),
        out_shape=jax.ShapeDtypeStruct((BC, P), jnp.float32),
    )(out_t)


@jax.jit
def kernel(x, index):
    xt = x.reshape(BC, H, W).transpose(1, 2, 0).reshape(H * W, BC)
    mesh = plsc.VectorSubcoreMesh(
        core_axis_name="c", subcore_axis_name="s", num_cores=NC, num_subcores=NS
    )
    out_t = pl.kernel(
        _sc_body,
        out_type=jax.ShapeDtypeStruct((B, P, C), jnp.float32),
        mesh=mesh,
        scratch_types=[
            pltpu.VMEM((8, PPW), jnp.float32),
            pltpu.VMEM((2, 4 * L, BC), jnp.float32),
            pltpu.VMEM((2, CHUNK, BC), jnp.float32),
            [pltpu.SemaphoreType.DMA, pltpu.SemaphoreType.DMA],
            [pltpu.SemaphoreType.DMA, pltpu.SemaphoreType.DMA],
        ],
    )(xt, index)
    return out_t.transpose(0, 2, 1).reshape(B, C, 1, P)


# speedup vs baseline: 1.5567x; 1.0050x over previous
import jax
import jax.numpy as jnp
from jax import lax
from jax.experimental import pallas as pl
from jax.experimental.pallas import tpu as pltpu
from jax.experimental.pallas import tpu_sc as plsc

B, C, H, W = 4, 128, 256, 512
P = 49152
BC = B * C
NC, NS, L = 2, 16, 16
NW = NC * NS
PPW = P // NW
CHUNK = 16
NCHUNK = PPW // CHUNK
NSLICE = BC // L


UNROLL = 8


def _sc_body(xt_hbm, index_hbm, out_hbm, ibuf, gbuf, obuf, gsems, osems):
    wid = lax.axis_index("s") * NC + lax.axis_index("c")
    base = wid * PPW

    pltpu.sync_copy(index_hbm.at[:, pl.ds(base, PPW)], ibuf)

    def fire_gather(g, slot):
        o = g * CHUNK
        x0 = ibuf[0, pl.ds(o, L)].astype(jnp.int32)
        y0 = ibuf[1, pl.ds(o, L)].astype(jnp.int32)
        x1 = ibuf[2, pl.ds(o, L)].astype(jnp.int32)
        y1 = ibuf[3, pl.ds(o, L)].astype(jnp.int32)
        r0 = y0 * W
        r1 = y1 * W
        sem = gsems[slot]
        pltpu.async_copy(xt_hbm.at[r0 + x0], gbuf.at[slot, pl.ds(0, L)], sem)
        pltpu.async_copy(xt_hbm.at[r1 + x0], gbuf.at[slot, pl.ds(L, L)], sem)
        pltpu.async_copy(xt_hbm.at[r0 + x1], gbuf.at[slot, pl.ds(2 * L, L)], sem)
        pltpu.async_copy(xt_hbm.at[r1 + x1], gbuf.at[slot, pl.ds(3 * L, L)], sem)

    def wait_gather(slot):
        pltpu.make_async_copy(
            xt_hbm.at[pl.ds(0, 4 * L)], gbuf.at[slot], gsems[slot]
        ).wait()

    def drain_out(slot):
        for b in range(B):
            pltpu.make_async_copy(
                obuf.at[slot, :, pl.ds(b * C, C)],
                out_hbm.at[b, pl.ds(0, CHUNK)],
                osems[slot],
            ).wait()

    def do_chunk(g, k, slot):
        wait_gather(slot)
        @pl.when(k > 0)
        def _():
            drain_out(slot)

        o = g * CHUNK
        wav = ibuf[4, pl.ds(o, L)]
        wbv = ibuf[5, pl.ds(o, L)]
        wcv = ibuf[6, pl.ds(o, L)]
        wdv = ibuf[7, pl.ds(o, L)]

        for i in range(CHUNK):
            wa = wav[i]
            wb = wbv[i]
            wc = wcv[i]
            wd = wdv[i]

            def blend(s, _):
                for u in range(UNROLL):
                    col = (s * UNROLL + u) * L
                    va = gbuf[slot, i, pl.ds(col, L)]
                    vb = gbuf[slot, L + i, pl.ds(col, L)]
                    vc = gbuf[slot, 2 * L + i, pl.ds(col, L)]
                    vd = gbuf[slot, 3 * L + i, pl.ds(col, L)]
                    obuf[slot, i, pl.ds(col, L)] = (
                        wa * va + wb * vb + wc * vc + wd * vd
                    )
                return 0

            lax.fori_loop(0, NSLICE // UNROLL, blend, 0)

        for b in range(B):
            pltpu.async_copy(
                obuf.at[slot, :, pl.ds(b * C, C)],
                out_hbm.at[b, pl.ds(base + o, CHUNK)],
                osems[slot],
            )
        fire_gather(jnp.minimum(g + 2, NCHUNK - 1), slot)

    fire_gather(0, 0)
    fire_gather(1, 1)

    def body(k, _):
        g = 2 * k
        do_chunk(g, k, 0)
        do_chunk(g + 1, k, 1)
        return 0

    lax.fori_loop(0, NCHUNK // 2, body, 0)
    wait_gather(0)
    wait_gather(1)
    drain_out(0)
    drain_out(1)


@jax.jit
def kernel(x, index):
    xt = x.reshape(BC, H, W).transpose(1, 2, 0).reshape(H * W, BC)
    mesh = plsc.VectorSubcoreMesh(
        core_axis_name="c", subcore_axis_name="s", num_cores=NC, num_subcores=NS
    )
    out_t = pl.kernel(
        _sc_body,
        out_type=jax.ShapeDtypeStruct((B, P, C), jnp.float32),
        mesh=mesh,
        scratch_types=[
            pltpu.VMEM((8, PPW), jnp.float32),
            pltpu.VMEM((2, 4 * L, BC), jnp.float32),
            pltpu.VMEM((2, CHUNK, BC), jnp.float32),
            [pltpu.SemaphoreType.DMA, pltpu.SemaphoreType.DMA],
            [pltpu.SemaphoreType.DMA, pltpu.SemaphoreType.DMA],
        ],
    )(xt, index)
    return out_t.transpose(0, 2, 1).reshape(B, C, 1, P)
